# baseline (speedup 1.0000x reference)
import functools

import jax
import jax.numpy as jnp
from jax import lax
from jax.experimental import pallas as pl
from jax.experimental.pallas import tpu as pltpu
from jax.experimental.pallas import tpu_sc as plsc

D_MODEL = 768
LN_EPS = 1e-12


def _word_gather_sc(ids, word_emb, *, batch, seq_len, q_off, span):
    info = plsc.get_sparse_core_info()
    n_workers = info.num_cores * info.num_subcores
    n_out = batch * span
    tpw = n_out // n_workers
    wpb = n_workers // batch
    chunk = 64
    n_chunks = tpw // chunk
    mesh = plsc.VectorSubcoreMesh(core_axis_name="c", subcore_axis_name="s")

    @functools.partial(
        pl.kernel,
        out_type=jax.ShapeDtypeStruct((n_out, D_MODEL), jnp.float32),
        mesh=mesh,
        scratch_types=[
            pltpu.VMEM((tpw,), jnp.int32),
            pltpu.VMEM((2, chunk, D_MODEL), jnp.float32),
            pltpu.SemaphoreType.DMA,
            pltpu.SemaphoreType.DMA,
            pltpu.SemaphoreType.DMA,
            pltpu.SemaphoreType.DMA,
        ],
    )
    def gather_kernel(ids_hbm, word_hbm, out_hbm, idx_v, rows_v,
                      sem_g0, sem_g1, sem_w0, sem_w1):
        wid = lax.axis_index("s") * info.num_cores + lax.axis_index("c")
        b = wid // wpb
        k = wid % wpb
        ids_base = b * seq_len + q_off + k * tpw
        out_base = wid * tpw
        sems_g = (sem_g0, sem_g1)
        sems_w = (sem_w0, sem_w1)
        pltpu.sync_copy(ids_hbm.at[pl.ds(ids_base, tpw)], idx_v)

        def start_gather(c, buf):
            return pltpu.async_copy(
                word_hbm.at[idx_v.at[pl.ds(c * chunk, chunk)]],
                rows_v.at[buf], sems_g[buf])

        def start_write(c, buf):
            return pltpu.async_copy(
                rows_v.at[buf], out_hbm.at[pl.ds(out_base + c * chunk, chunk)],
                sems_w[buf])

        gathers = [start_gather(0, 0), None]
        writes = [None, None]
        for c in range(n_chunks):
            buf = c % 2
            gathers[buf].wait()
            writes[buf] = start_write(c, buf)
            nc = c + 1
            if nc < n_chunks:
                nbuf = nc % 2
                if writes[nbuf] is not None:
                    writes[nbuf].wait()
                gathers[nbuf] = start_gather(nc, nbuf)
        for buf in (0, 1):
            if writes[buf] is not None:
                writes[buf].wait()

    return gather_kernel(ids, word_emb)


POS_BLK = 512


def _add_ln_tc(x, tts3, pos_emb, type_emb, gamma, beta, out_buf, *,
               q_off, span, batch, seq_len, n_total):
    nb = span // POS_BLK
    q0 = q_off // POS_BLK
    s_blocks = seq_len // POS_BLK

    def body(x_ref, tt_ref, pos_ref, typ_ref, g_ref, b_ref, *rest):
        o_ref = rest[-1]
        ttf = tt_ref[0].astype(jnp.float32).reshape(POS_BLK, 1)
        t0 = typ_ref[0:1, :]
        t1 = typ_ref[1:2, :]
        xb = x_ref[...] + pos_ref[...] + t0 + ttf * (t1 - t0)
        mean = jnp.mean(xb, axis=-1, keepdims=True)
        xc = xb - mean
        var = jnp.mean(xc * xc, axis=-1, keepdims=True)
        o_ref[...] = xc * lax.rsqrt(var + LN_EPS) * g_ref[...] + b_ref[...]

    in_specs = [
        pl.BlockSpec((POS_BLK, D_MODEL), lambda jj, i, nb=nb: (i * nb + jj, 0)),
        pl.BlockSpec((1, 1, POS_BLK), lambda jj, i, q0=q0: (i, 0, q0 + jj)),
        pl.BlockSpec((POS_BLK, D_MODEL), lambda jj, i, q0=q0: (q0 + jj, 0)),
        pl.BlockSpec((2, D_MODEL), lambda jj, i: (0, 0)),
        pl.BlockSpec((1, D_MODEL), lambda jj, i: (0, 0)),
        pl.BlockSpec((1, D_MODEL), lambda jj, i: (0, 0)),
    ]
    args = [x, tts3, pos_emb, type_emb, gamma.reshape(1, D_MODEL),
            beta.reshape(1, D_MODEL)]
    aliases = {}
    if out_buf is not None:
        in_specs.append(pl.BlockSpec(memory_space=pl.ANY))
        args.append(out_buf)
        aliases = {6: 0}

    return pl.pallas_call(
        body,
        grid=(nb, batch),
        in_specs=in_specs,
        out_specs=pl.BlockSpec(
            (POS_BLK, D_MODEL),
            lambda jj, i, q0=q0, s_blocks=s_blocks: (i * s_blocks + q0 + jj, 0)),
        out_shape=jax.ShapeDtypeStruct((n_total, D_MODEL), jnp.float32),
        input_output_aliases=aliases,
    )(*args)


def kernel(input_ids, token_type_ids, word_emb, pos_emb, type_emb, ln_gamma, ln_beta):
    b, s = input_ids.shape
    n = b * s
    spans = ((0, 1024), (1024, 512), (1536, 512))
    ids = input_ids.reshape(n).astype(jnp.int32)
    tts3 = token_type_ids.reshape(b, 1, s).astype(jnp.int32)

    gathers = [
        _word_gather_sc(ids, word_emb, batch=b, seq_len=s, q_off=q_off,
                        span=span)
        for q_off, span in spans
    ]

    buf = None
    for (q_off, span), g in zip(spans, gathers):
        buf = _add_ln_tc(g, tts3, pos_emb, type_emb,
                         ln_gamma, ln_beta, buf, q_off=q_off, span=span,
                         batch=b, seq_len=s, n_total=n)
    return buf.reshape(b, s, D_MODEL)


# speedup vs baseline: 1.0181x; 1.0181x over previous
import functools

import jax
import jax.numpy as jnp
from jax import lax
from jax.experimental import pallas as pl
from jax.experimental.pallas import tpu as pltpu
from jax.experimental.pallas import tpu_sc as plsc

D_MODEL = 768
LN_EPS = 1e-12


def _word_gather_sc(ids, word_emb, *, batch, seq_len, q_off, span):
    info = plsc.get_sparse_core_info()
    n_workers = info.num_cores * info.num_subcores
    n_out = batch * span
    tpw = n_out // n_workers
    wpb = n_workers // batch
    chunk = 64
    n_chunks = tpw // chunk
    mesh = plsc.VectorSubcoreMesh(core_axis_name="c", subcore_axis_name="s")

    @functools.partial(
        pl.kernel,
        out_type=jax.ShapeDtypeStruct((n_out, D_MODEL), jnp.float32),
        mesh=mesh,
        scratch_types=[
            pltpu.VMEM((tpw,), jnp.int32),
            pltpu.VMEM((2, chunk, D_MODEL), jnp.float32),
            pltpu.SemaphoreType.DMA,
            pltpu.SemaphoreType.DMA,
            pltpu.SemaphoreType.DMA,
            pltpu.SemaphoreType.DMA,
        ],
    )
    def gather_kernel(ids_hbm, word_hbm, out_hbm, idx_v, rows_v,
                      sem_g0, sem_g1, sem_w0, sem_w1):
        wid = lax.axis_index("s") * info.num_cores + lax.axis_index("c")
        b = wid // wpb
        k = wid % wpb
        ids_base = b * seq_len + q_off + k * tpw
        out_base = wid * tpw
        sems_g = (sem_g0, sem_g1)
        sems_w = (sem_w0, sem_w1)
        pltpu.sync_copy(ids_hbm.at[pl.ds(ids_base, tpw)], idx_v)

        def start_gather(c, buf):
            return pltpu.async_copy(
                word_hbm.at[idx_v.at[pl.ds(c * chunk, chunk)]],
                rows_v.at[buf], sems_g[buf])

        def start_write(c, buf):
            return pltpu.async_copy(
                rows_v.at[buf], out_hbm.at[pl.ds(out_base + c * chunk, chunk)],
                sems_w[buf])

        gathers = [start_gather(0, 0), None]
        writes = [None, None]
        for c in range(n_chunks):
            buf = c % 2
            gathers[buf].wait()
            writes[buf] = start_write(c, buf)
            nc = c + 1
            if nc < n_chunks:
                nbuf = nc % 2
                if writes[nbuf] is not None:
                    writes[nbuf].wait()
                gathers[nbuf] = start_gather(nc, nbuf)
        for buf in (0, 1):
            if writes[buf] is not None:
                writes[buf].wait()

    return gather_kernel(ids, word_emb)


def _add_ln_tc(x, tts3, pos_emb, type_emb, gamma, beta, out_buf, *,
               q_off, span, batch, seq_len, n_total):
    blk = min(span, 1024)
    nb = span // blk
    q0 = q_off // blk
    s_blocks = seq_len // blk

    def body(x_ref, tt_ref, pos_ref, typ_ref, g_ref, b_ref, *rest):
        o_ref = rest[-1]
        ttf = tt_ref[0].astype(jnp.float32).reshape(blk, 1)
        t0 = typ_ref[0:1, :]
        t1 = typ_ref[1:2, :]
        xb = x_ref[...] + pos_ref[...] + t0 + ttf * (t1 - t0)
        mean = jnp.mean(xb, axis=-1, keepdims=True)
        xc = xb - mean
        var = jnp.mean(xc * xc, axis=-1, keepdims=True)
        o_ref[...] = xc * lax.rsqrt(var + LN_EPS) * g_ref[...] + b_ref[...]

    in_specs = [
        pl.BlockSpec((blk, D_MODEL), lambda jj, i, nb=nb: (i * nb + jj, 0)),
        pl.BlockSpec((1, 1, blk), lambda jj, i, q0=q0: (i, 0, q0 + jj)),
        pl.BlockSpec((blk, D_MODEL), lambda jj, i, q0=q0: (q0 + jj, 0)),
        pl.BlockSpec((2, D_MODEL), lambda jj, i: (0, 0)),
        pl.BlockSpec((1, D_MODEL), lambda jj, i: (0, 0)),
        pl.BlockSpec((1, D_MODEL), lambda jj, i: (0, 0)),
    ]
    args = [x, tts3, pos_emb, type_emb, gamma.reshape(1, D_MODEL),
            beta.reshape(1, D_MODEL)]
    aliases = {}
    if out_buf is not None:
        in_specs.append(pl.BlockSpec(memory_space=pl.ANY))
        args.append(out_buf)
        aliases = {6: 0}

    return pl.pallas_call(
        body,
        grid=(nb, batch),
        in_specs=in_specs,
        out_specs=pl.BlockSpec(
            (blk, D_MODEL),
            lambda jj, i, q0=q0, s_blocks=s_blocks: (i * s_blocks + q0 + jj, 0)),
        out_shape=jax.ShapeDtypeStruct((n_total, D_MODEL), jnp.float32),
        input_output_aliases=aliases,
    )(*args)


def kernel(input_ids, token_type_ids, word_emb, pos_emb, type_emb, ln_gamma, ln_beta):
    b, s = input_ids.shape
    n = b * s
    spans = ((0, 1024), (1024, 512), (1536, 512))
    ids = input_ids.reshape(n).astype(jnp.int32)
    tts3 = token_type_ids.reshape(b, 1, s).astype(jnp.int32)

    gathers = [
        _word_gather_sc(ids, word_emb, batch=b, seq_len=s, q_off=q_off,
                        span=span)
        for q_off, span in spans
    ]

    buf = None
    for (q_off, span), g in zip(spans, gathers):
        buf = _add_ln_tc(g, tts3, pos_emb, type_emb,
                         ln_gamma, ln_beta, buf, q_off=q_off, span=span,
                         batch=b, seq_len=s, n_total=n)
    return buf.reshape(b, s, D_MODEL)


# speedup vs baseline: 1.0310x; 1.0126x over previous
import functools

import jax
import jax.numpy as jnp
from jax import lax
from jax.experimental import pallas as pl
from jax.experimental.pallas import tpu as pltpu
from jax.experimental.pallas import tpu_sc as plsc

D_MODEL = 768
LN_EPS = 1e-12


def _word_gather_sc(ids, word_emb, *, batch, seq_len, q_off, span):
    info = plsc.get_sparse_core_info()
    n_workers = info.num_cores * info.num_subcores
    n_out = batch * span
    tpw = n_out // n_workers
    wpb = n_workers // batch
    chunk = 64
    n_chunks = tpw // chunk
    mesh = plsc.VectorSubcoreMesh(core_axis_name="c", subcore_axis_name="s")

    @functools.partial(
        pl.kernel,
        out_type=jax.ShapeDtypeStruct((n_out, D_MODEL), jnp.float32),
        mesh=mesh,
        scratch_types=[
            pltpu.VMEM((tpw,), jnp.int32),
            pltpu.VMEM((2, chunk, D_MODEL), jnp.float32),
            pltpu.SemaphoreType.DMA,
            pltpu.SemaphoreType.DMA,
            pltpu.SemaphoreType.DMA,
            pltpu.SemaphoreType.DMA,
        ],
    )
    def gather_kernel(ids_hbm, word_hbm, out_hbm, idx_v, rows_v,
                      sem_g0, sem_g1, sem_w0, sem_w1):
        wid = lax.axis_index("s") * info.num_cores + lax.axis_index("c")
        b = wid // wpb
        k = wid % wpb
        ids_base = b * seq_len + q_off + k * tpw
        out_base = wid * tpw
        sems_g = (sem_g0, sem_g1)
        sems_w = (sem_w0, sem_w1)
        pltpu.sync_copy(ids_hbm.at[pl.ds(ids_base, tpw)], idx_v)

        def start_gather(c, buf):
            return pltpu.async_copy(
                word_hbm.at[idx_v.at[pl.ds(c * chunk, chunk)]],
                rows_v.at[buf], sems_g[buf])

        def start_write(c, buf):
            return pltpu.async_copy(
                rows_v.at[buf], out_hbm.at[pl.ds(out_base + c * chunk, chunk)],
                sems_w[buf])

        gathers = [start_gather(0, 0), None]
        writes = [None, None]
        for c in range(n_chunks):
            buf = c % 2
            gathers[buf].wait()
            writes[buf] = start_write(c, buf)
            nc = c + 1
            if nc < n_chunks:
                nbuf = nc % 2
                if writes[nbuf] is not None:
                    writes[nbuf].wait()
                gathers[nbuf] = start_gather(nc, nbuf)
        for buf in (0, 1):
            if writes[buf] is not None:
                writes[buf].wait()

    return gather_kernel(ids, word_emb)


POS_BLK = 1024


def _add_ln_tc(x, tts3, pos_emb, type_emb, gamma, beta, out_buf, *,
               q_off, span, batch, seq_len, n_total):
    nb = span // POS_BLK
    q0 = q_off // POS_BLK
    s_blocks = seq_len // POS_BLK

    def body(x_ref, tt_ref, pos_ref, typ_ref, g_ref, b_ref, *rest):
        o_ref = rest[-1]
        ttf = tt_ref[0].astype(jnp.float32).reshape(POS_BLK, 1)
        t0 = typ_ref[0:1, :]
        t1 = typ_ref[1:2, :]
        xb = x_ref[...] + pos_ref[...] + t0 + ttf * (t1 - t0)
        mean = jnp.mean(xb, axis=-1, keepdims=True)
        xc = xb - mean
        var = jnp.mean(xc * xc, axis=-1, keepdims=True)
        o_ref[...] = xc * lax.rsqrt(var + LN_EPS) * g_ref[...] + b_ref[...]

    in_specs = [
        pl.BlockSpec((POS_BLK, D_MODEL), lambda jj, i, nb=nb: (i * nb + jj, 0)),
        pl.BlockSpec((1, 1, POS_BLK), lambda jj, i, q0=q0: (i, 0, q0 + jj)),
        pl.BlockSpec((POS_BLK, D_MODEL), lambda jj, i, q0=q0: (q0 + jj, 0)),
        pl.BlockSpec((2, D_MODEL), lambda jj, i: (0, 0)),
        pl.BlockSpec((1, D_MODEL), lambda jj, i: (0, 0)),
        pl.BlockSpec((1, D_MODEL), lambda jj, i: (0, 0)),
    ]
    args = [x, tts3, pos_emb, type_emb, gamma.reshape(1, D_MODEL),
            beta.reshape(1, D_MODEL)]
    aliases = {}
    if out_buf is not None:
        in_specs.append(pl.BlockSpec(memory_space=pl.ANY))
        args.append(out_buf)
        aliases = {6: 0}

    return pl.pallas_call(
        body,
        grid=(nb, batch),
        in_specs=in_specs,
        out_specs=pl.BlockSpec(
            (POS_BLK, D_MODEL),
            lambda jj, i, q0=q0, s_blocks=s_blocks: (i * s_blocks + q0 + jj, 0)),
        out_shape=jax.ShapeDtypeStruct((n_total, D_MODEL), jnp.float32),
        input_output_aliases=aliases,
    )(*args)


def kernel(input_ids, token_type_ids, word_emb, pos_emb, type_emb, ln_gamma, ln_beta):
    b, s = input_ids.shape
    n = b * s
    spans = ((0, 1024), (1024, 1024))
    ids = input_ids.reshape(n).astype(jnp.int32)
    tts3 = token_type_ids.reshape(b, 1, s).astype(jnp.int32)

    gathers = [
        _word_gather_sc(ids, word_emb, batch=b, seq_len=s, q_off=q_off,
                        span=span)
        for q_off, span in spans
    ]

    buf = None
    for (q_off, span), g in zip(spans, gathers):
        buf = _add_ln_tc(g, tts3, pos_emb, type_emb,
                         ln_gamma, ln_beta, buf, q_off=q_off, span=span,
                         batch=b, seq_len=s, n_total=n)
    return buf.reshape(b, s, D_MODEL)
